# baseline (speedup 1.0000x reference)
import functools

import jax
import jax.numpy as jnp
from jax import lax
from jax.experimental import pallas as pl
from jax.experimental.pallas import tpu as pltpu
from jax.experimental.pallas import tpu_sc as plsc

_B = 16384
_D = 64
_NW = 32
_BPW = _B // _NW
_RCH = 128
_NCH = _BPW // _RCH
_RING = 8
_L = 16


def _build_sc_gather():
    mesh = plsc.VectorSubcoreMesh(core_axis_name="c", subcore_axis_name="s")

    @functools.partial(
        pl.kernel,
        out_type=(
            jax.ShapeDtypeStruct((_B, _D), jnp.float32),
            jax.ShapeDtypeStruct((_B, _D), jnp.float32),
        ),
        mesh=mesh,
        scratch_types=[
            pltpu.VMEM((_BPW, 2), jnp.int32),
            pltpu.VMEM((_RCH, _D), jnp.float32),
            pltpu.VMEM((_RCH, _D), jnp.float32),
            pltpu.SemaphoreType.DMA((_RING,)),
            pltpu.SemaphoreType.DMA((_RING,)),
        ],
        compiler_params=pltpu.CompilerParams(needs_layout_passes=False),
    )
    def gather(x_hbm, utab_hbm, itab_hbm, ux_hbm, ix_hbm,
               x_v, urow_v, irow_v, usem, isem):
        wid = lax.axis_index("s") * 2 + lax.axis_index("c")
        base = wid * _BPW
        pltpu.sync_copy(x_hbm.at[pl.ds(base, _BPW)], x_v)

        iota = jnp.arange(_L, dtype=jnp.int32)
        zero = jnp.zeros((_L,), dtype=jnp.int32)
        one = jnp.ones((_L,), dtype=jnp.int32)

        def fire(tab_hbm, row_v, sem, idx_s, r):
            pltpu.make_async_copy(
                tab_hbm.at[pl.ds(idx_s, 1)],
                row_v.at[pl.ds(r, 1)],
                sem.at[r % _RING]).start()

        def drain(tab_hbm, row_v, sem, r):
            pltpu.make_async_copy(
                tab_hbm.at[pl.ds(0, 1)],
                row_v.at[pl.ds(r, 1)],
                sem.at[r % _RING]).wait()

        def chunk_body(ch, _):
            off = ch * _RCH
            for g in range(_RCH // _L):
                rows16 = off + g * _L + iota
                u16 = plsc.load_gather(x_v, [rows16, zero])
                i16 = plsc.load_gather(x_v, [rows16, one])
                for j in range(_L):
                    r = g * _L + j
                    us = jnp.max(jnp.where(iota == j, u16, zero))
                    is_ = jnp.max(jnp.where(iota == j, i16, zero))
                    if r >= _RING:
                        drain(utab_hbm, urow_v, usem, r - _RING)
                        drain(itab_hbm, irow_v, isem, r - _RING)
                    fire(utab_hbm, urow_v, usem, us, r)
                    fire(itab_hbm, irow_v, isem, is_, r)
            for r in range(_RCH - _RING, _RCH):
                drain(utab_hbm, urow_v, usem, r)
                drain(itab_hbm, irow_v, isem, r)
            pltpu.sync_copy(urow_v, ux_hbm.at[pl.ds(base + off, _RCH)])
            pltpu.sync_copy(irow_v, ix_hbm.at[pl.ds(base + off, _RCH)])
            return 0

        lax.fori_loop(0, _NCH, chunk_body, 0)

    return gather


_sc_gather = _build_sc_gather()

_MM_BLK = 1024


def _mm_body(ux_ref, ix_ref, w_ref, b_ref, o_ref):
    w = w_ref[...]
    acc = jnp.dot(ux_ref[...], w[:, :_D].T, preferred_element_type=jnp.float32)
    acc = acc + jnp.dot(ix_ref[...], w[:, _D:].T, preferred_element_type=jnp.float32)
    o_ref[...] = acc + b_ref[...]


def _tc_matmul(ux, ix, w, b):
    return pl.pallas_call(
        _mm_body,
        grid=(_B // _MM_BLK,),
        in_specs=[
            pl.BlockSpec((_MM_BLK, _D), lambda i: (i, 0)),
            pl.BlockSpec((_MM_BLK, _D), lambda i: (i, 0)),
            pl.BlockSpec((_D, 2 * _D), lambda i: (0, 0)),
            pl.BlockSpec((_D,), lambda i: (0,)),
        ],
        out_specs=pl.BlockSpec((_MM_BLK, _D), lambda i: (i, 0)),
        out_shape=jax.ShapeDtypeStruct((_B, _D), jnp.float32),
    )(ux, ix, w, b)


def kernel(x, user_table, item_table, W, b):
    ux, ix = _sc_gather(x, user_table, item_table)
    return _tc_matmul(ux, ix, W, b)
